# baseline (speedup 1.0000x reference)
import functools
import jax
import jax.numpy as jnp
from jax import lax
from jax.experimental import pallas as pl
from jax.experimental.pallas import tpu as pltpu
from jax.experimental.pallas import tpu_sc as plsc

N = 10000
E = 320000
D = 128
D_OUT = 40

NC = 2
NS = 16
NW = NC * NS
EPW = E // NW
CHUNK = 128
NPAD = 10240
EPW_PAD = NPAD
NCHUNK = EPW_PAD // CHUNK
ACH = 80
ANCH = EPW_PAD // ACH
RPT = NPAD // NS
WCHUNK = 128
DR = NPAD // D


def _mesh():
  return plsc.VectorSubcoreMesh(core_axis_name="c", subcore_axis_name="s",
                                num_cores=NC, num_subcores=NS)


@functools.lru_cache(maxsize=None)
def _sc_agg():
  scratch = [
      pltpu.VMEM((ACH,), jnp.int32),
      pltpu.VMEM((ACH,), jnp.int32),
      pltpu.VMEM((ACH,), jnp.int32),
      pltpu.VMEM((ACH,), jnp.int32),
      pltpu.VMEM((ACH, D), jnp.float32),
      pltpu.VMEM((ACH, D), jnp.float32),
      pltpu.VMEM((WCHUNK, D), jnp.float32),
      pltpu.VMEM_SHARED((NPAD, D), jnp.float32),
      pltpu.SemaphoreType.DMA,
      pltpu.SemaphoreType.DMA,
      pltpu.SemaphoreType.DMA,
      pltpu.SemaphoreType.DMA,
  ]

  @functools.partial(
      pl.kernel, out_type=jax.ShapeDtypeStruct((NC, NPAD, D), jnp.float32),
      mesh=_mesh(), scratch_types=scratch,
      compiler_params=pltpu.CompilerParams(needs_layout_passes=False))
  def agg(h_hbm, src_hbm, dst_hbm, out_hbm,
          src0_v, src1_v, dst0_v, dst1_v, rows0_v, rows1_v, buf_v, acc_sh,
          semr0, semr1, semi0, semi1):
    c = lax.axis_index("c")
    s = lax.axis_index("s")
    wid = s * NC + c
    base = wid * EPW_PAD

    def load_idx(t, sv, dv, semi):
      pltpu.async_copy(src_hbm.at[pl.ds(base + t * ACH, ACH)], sv, semi)
      pltpu.async_copy(dst_hbm.at[pl.ds(base + t * ACH, ACH)], dv, semi)

    def wait_idx(sv, dv, semi):
      pltpu.make_async_copy(src_hbm.at[pl.ds(base, ACH)], sv, semi).wait()
      pltpu.make_async_copy(dst_hbm.at[pl.ds(base, ACH)], dv, semi).wait()

    def gather(sv, rows, semr):
      pltpu.async_copy(h_hbm.at[sv], rows, semr)

    def drain(rows, semr):
      pltpu.make_async_copy(h_hbm.at[src0_v], rows, semr).wait()

    def scatter(dv, rows):
      pltpu.sync_copy(rows, acc_sh.at[dv], add=True)

    load_idx(0, src0_v, dst0_v, semi0)

    zeros16 = jnp.zeros((16,), jnp.float32)

    def zrow(r, _):
      for j in range(D // 16):
        buf_v[r, pl.ds(j * 16, 16)] = zeros16
      return 0

    lax.fori_loop(0, WCHUNK, zrow, 0)
    row0 = s * RPT
    for k in range(RPT // WCHUNK):
      pltpu.sync_copy(buf_v, acc_sh.at[pl.ds(row0 + k * WCHUNK, WCHUNK)])
    plsc.subcore_barrier()

    wait_idx(src0_v, dst0_v, semi0)

    def step(t, _):
      pltpu.async_copy(h_hbm.at[src0_v], rows0_v, semr0).wait()
      pltpu.sync_copy(rows0_v, acc_sh.at[dst0_v], add=True)
      pltpu.sync_copy(src_hbm.at[pl.ds(base + (t + 1) * ACH, ACH)], src0_v)
      pltpu.sync_copy(dst_hbm.at[pl.ds(base + (t + 1) * ACH, ACH)], dst0_v)
      return 0

    lax.fori_loop(0, ANCH - 1, step, 0)
    pltpu.async_copy(h_hbm.at[src0_v], rows0_v, semr0).wait()
    pltpu.sync_copy(rows0_v, acc_sh.at[dst0_v], add=True)
    del gather, drain, scatter, semr1, semi1, src1_v, dst1_v, rows1_v
    plsc.subcore_barrier()

    for k in range(RPT // WCHUNK):
      r0 = row0 + k * WCHUNK
      pltpu.sync_copy(acc_sh.at[pl.ds(r0, WCHUNK)], buf_v)
      pltpu.sync_copy(buf_v, out_hbm.at[c, pl.ds(r0, WCHUNK)])

  return agg


@functools.lru_cache(maxsize=None)
def _sc_deg():
  scratch = [
      pltpu.VMEM((NCHUNK, CHUNK), jnp.int32),
      pltpu.VMEM((NPAD,), jnp.float32),
      pltpu.VMEM((DR, D), jnp.float32),
      pltpu.VMEM((DR,), jnp.int32),
      pltpu.VMEM_SHARED((DR, D), jnp.float32),
      pltpu.SemaphoreType.DMA,
  ]

  @functools.partial(
      pl.kernel, out_type=jax.ShapeDtypeStruct((NC, DR, D), jnp.float32),
      mesh=_mesh(), scratch_types=scratch,
      compiler_params=pltpu.CompilerParams(needs_layout_passes=False))
  def deg(dst_hbm, out_hbm, dst_v, deg_v, deg2_v, iota_v, deg_sh, sem):
    c = lax.axis_index("c")
    s = lax.axis_index("s")
    wid = s * NC + c

    pltpu.async_copy(dst_hbm.at[wid], dst_v, sem)
    zeros16 = jnp.zeros((16,), jnp.float32)
    ones16 = jnp.ones((16,), jnp.float32)

    def zdeg(i, _):
      deg_v[pl.ds(i * 16, 16)] = zeros16
      return 0

    lax.fori_loop(0, NPAD // 16, zdeg, 0)
    for r in range(8):
      for j in range(D // 16):
        deg2_v[r, pl.ds(j * 16, 16)] = zeros16
    iota16 = lax.iota(jnp.int32, 16)
    for i in range(DR // 16):
      iota_v[pl.ds(i * 16, 16)] = iota16 + (i * 16)
    @pl.when(s < DR // 8)
    def _():
      pltpu.sync_copy(deg2_v.at[pl.ds(0, 8)], deg_sh.at[pl.ds(s * 8, 8)])
    pltpu.make_async_copy(dst_hbm.at[wid], dst_v, sem).wait()

    def step(t, _):
      for j in range(CHUNK // 16):
        idx = dst_v[t, pl.ds(j * 16, 16)]
        plsc.addupdate_scatter(deg_v, [idx], ones16)
      return 0

    lax.fori_loop(0, NCHUNK, step, 0)

    def stage(r, _):
      for j in range(D // 16):
        deg2_v[r, pl.ds(j * 16, 16)] = deg_v[pl.ds(r * D + j * 16, 16)]
      return 0

    lax.fori_loop(0, DR, stage, 0)
    plsc.subcore_barrier()
    pltpu.sync_copy(deg2_v, deg_sh.at[iota_v], add=True)
    plsc.subcore_barrier()

    @pl.when(s < DR // 8)
    def _():
      pltpu.sync_copy(deg_sh.at[pl.ds(s * 8, 8)], deg2_v.at[pl.ds(0, 8)])
      pltpu.sync_copy(deg2_v.at[pl.ds(0, 8)], out_hbm.at[c, pl.ds(s * 8, 8)])

  return deg


BR = 2000


def _make_comb_body(relu: bool):
  def body(h_ref, s0_ref, s1_ref, d0_ref, d1_ref, ws_ref, wn_ref, b_ref,
           o_ref):
    inv = 1.0 / jnp.maximum(d0_ref[...] + d1_ref[...], 1.0)
    hn = (s0_ref[...] + s1_ref[...]) * inv
    h = h_ref[...]
    o = (jnp.dot(h, ws_ref[...], preferred_element_type=jnp.float32)
         + jnp.dot(hn, wn_ref[...], preferred_element_type=jnp.float32)
         + b_ref[...])
    o_ref[...] = jnp.maximum(o, 0.0) if relu else o
  return body


def _tc_comb(h, s0, s1, d0, d1, ws, wn, b, relu):
  n, d = h.shape
  do = ws.shape[1]
  return pl.pallas_call(
      _make_comb_body(relu),
      grid=(n // BR,),
      in_specs=[
          pl.BlockSpec((BR, d), lambda i: (i, 0)),
          pl.BlockSpec((BR, d), lambda i: (i, 0)),
          pl.BlockSpec((BR, d), lambda i: (i, 0)),
          pl.BlockSpec((BR, 1), lambda i: (i, 0)),
          pl.BlockSpec((BR, 1), lambda i: (i, 0)),
          pl.BlockSpec((d, do), lambda i: (0, 0)),
          pl.BlockSpec((d, do), lambda i: (0, 0)),
          pl.BlockSpec((1, do), lambda i: (0, 0)),
      ],
      out_specs=pl.BlockSpec((BR, do), lambda i: (i, 0)),
      out_shape=jax.ShapeDtypeStruct((n, do), jnp.float32),
  )(h, s0, s1, d0, d1, ws, wn, b)


def kernel(x, edge_index, edge_weight,
           W_self1, W_neigh1, b1,
           W_self2, W_neigh2, b2,
           W_self3, W_neigh3, b3):
  src = edge_index[0].astype(jnp.int32)
  dst = edge_index[1].astype(jnp.int32)

  pad = ((0, 0), (0, EPW_PAD - EPW))
  pad_dst = N + jnp.arange(EPW_PAD - EPW, dtype=jnp.int32) % (NPAD - N)
  src_p = jnp.pad(src.reshape(NW, EPW), pad).reshape(NW, NCHUNK, CHUNK)
  dst_p = jnp.concatenate(
      [dst.reshape(NW, EPW),
       jnp.broadcast_to(pad_dst, (NW, EPW_PAD - EPW))],
      axis=1).reshape(NW, NCHUNK, CHUNK)

  src_f = src_p.reshape(-1)
  dst_f = dst_p.reshape(-1)

  degp = _sc_deg()(dst_p)
  degf = degp.reshape(NC, NPAD)
  d0 = degf[0, :N].reshape(N, 1)
  d1 = degf[1, :N].reshape(N, 1)

  p1 = _sc_agg()(x, src_f, dst_f)
  h1 = _tc_comb(x, p1[0, :N], p1[1, :N], d0, d1,
                W_self1, W_neigh1, b1.reshape(1, -1), relu=True)
  p2 = _sc_agg()(h1, src_f, dst_f)
  h2 = _tc_comb(h1, p2[0, :N], p2[1, :N], d0, d1,
                W_self2, W_neigh2, b2.reshape(1, -1), relu=True)
  p3 = _sc_agg()(h2, src_f, dst_f)
  out = _tc_comb(h2, p3[0, :N], p3[1, :N], d0, d1,
                 W_self3, W_neigh3, b3.reshape(1, -1), relu=False)
  return out


# speedup vs baseline: 1.8054x; 1.8054x over previous
import functools
import jax
import jax.numpy as jnp
from jax import lax
from jax.experimental import pallas as pl
from jax.experimental.pallas import tpu as pltpu
from jax.experimental.pallas import tpu_sc as plsc

N = 10000
E = 320000
D = 128
D_OUT = 40

NC = 2
NS = 16
NW = NC * NS
EPW = E // NW
CHUNK = 80
NCHUNK = EPW // CHUNK
NPAD = 10240
RPT = NPAD // NS
WCHUNK = 128


@functools.lru_cache(maxsize=None)
def _sc_agg(with_deg: bool):
  mesh = plsc.VectorSubcoreMesh(core_axis_name="c", subcore_axis_name="s",
                                num_cores=NC, num_subcores=NS)

  out_type = jax.ShapeDtypeStruct((NC, NPAD, D), jnp.float32)
  scratch = [
      pltpu.VMEM((CHUNK,), jnp.int32),
      pltpu.VMEM((CHUNK,), jnp.int32),
      pltpu.VMEM((CHUNK, D), jnp.float32),
      pltpu.VMEM((WCHUNK, D), jnp.float32),
      pltpu.VMEM_SHARED((NPAD, D), jnp.float32),
      pltpu.SemaphoreType.DMA,
  ]
  DR = NPAD // D
  if with_deg:
    out_type = [out_type, jax.ShapeDtypeStruct((NC, DR, D), jnp.float32)]
    scratch.append(pltpu.VMEM((NPAD,), jnp.float32))
    scratch.append(pltpu.VMEM((DR, D), jnp.float32))
    scratch.append(pltpu.VMEM_SHARED((DR, D), jnp.float32))
    scratch.append(pltpu.VMEM((DR,), jnp.int32))
  else:
    scratch.append(pltpu.VMEM((CHUNK,), jnp.int32))
    scratch.append(pltpu.VMEM((CHUNK,), jnp.int32))
    scratch.append(pltpu.VMEM((CHUNK, D), jnp.float32))
    scratch.append(pltpu.SemaphoreType.DMA)
    scratch.append(pltpu.SemaphoreType.DMA)
    scratch.append(pltpu.SemaphoreType.DMA)

  @functools.partial(
      pl.kernel, out_type=out_type, mesh=mesh, scratch_types=scratch,
      compiler_params=pltpu.CompilerParams(needs_layout_passes=False))
  def agg(h_hbm, src_hbm, dst_hbm, out_hbm, *rest):
    if with_deg:
      (deg_hbm, src_v, dst_v, rows_v, buf_v, acc_sh, sem, deg_v,
       deg2_v, deg_sh, iota_v) = rest
    else:
      src_v, dst_v, rows_v, buf_v, acc_sh, sem, *_unused = rest
    c = lax.axis_index("c")
    s = lax.axis_index("s")
    wid = s * NC + c

    zeros16 = jnp.zeros((16,), jnp.float32)
    ones16 = jnp.ones((16,), jnp.float32)

    def zrow(r, _):
      for j in range(D // 16):
        buf_v[r, pl.ds(j * 16, 16)] = zeros16
      return 0

    lax.fori_loop(0, WCHUNK, zrow, 0)
    row0 = s * RPT
    for k in range(RPT // WCHUNK):
      pltpu.sync_copy(buf_v, acc_sh.at[pl.ds(row0 + k * WCHUNK, WCHUNK)])
    if with_deg:
      def zdeg(i, _):
        deg_v[pl.ds(i * 16, 16)] = zeros16
        return 0
      lax.fori_loop(0, NPAD // 16, zdeg, 0)
      iota16 = lax.iota(jnp.int32, 16)
      for i in range(DR // 16):
        iota_v[pl.ds(i * 16, 16)] = iota16 + (i * 16)
      @pl.when(s < DR // 8)
      def _():
        pltpu.sync_copy(buf_v.at[pl.ds(0, 8)], deg_sh.at[pl.ds(s * 8, 8)])
    plsc.subcore_barrier()

    def step(t, _):
      base = wid * EPW + t * CHUNK
      pltpu.sync_copy(src_hbm.at[pl.ds(base, CHUNK)], src_v)
      pltpu.sync_copy(dst_hbm.at[pl.ds(base, CHUNK)], dst_v)
      pltpu.async_copy(h_hbm.at[src_v], rows_v, sem).wait()
      if with_deg:
        for j in range(CHUNK // 16):
          idx = dst_v[pl.ds(j * 16, 16)]
          plsc.addupdate_scatter(deg_v, [idx], ones16)
      pltpu.sync_copy(rows_v, acc_sh.at[dst_v], add=True)
      return 0

    lax.fori_loop(0, NCHUNK, step, 0)
    plsc.subcore_barrier()

    if with_deg:
      def stage(r, _):
        for j in range(D // 16):
          deg2_v[r, pl.ds(j * 16, 16)] = deg_v[pl.ds(r * D + j * 16, 16)]
        return 0
      lax.fori_loop(0, DR, stage, 0)
      pltpu.sync_copy(deg2_v, deg_sh.at[iota_v], add=True)
      plsc.subcore_barrier()

    for k in range(RPT // WCHUNK):
      r0 = row0 + k * WCHUNK
      pltpu.sync_copy(acc_sh.at[pl.ds(r0, WCHUNK)], buf_v)
      pltpu.sync_copy(buf_v, out_hbm.at[c, pl.ds(r0, WCHUNK)])
    if with_deg:
      @pl.when(s < DR // 8)
      def _():
        pltpu.sync_copy(deg_sh.at[pl.ds(s * 8, 8)], deg2_v.at[pl.ds(0, 8)])
        pltpu.sync_copy(deg2_v.at[pl.ds(0, 8)],
                        deg_hbm.at[c, pl.ds(s * 8, 8)])

  return agg


BR = 2000


def _make_comb_body(relu: bool):
  def body(h_ref, s0_ref, s1_ref, d0_ref, d1_ref, ws_ref, wn_ref, b_ref,
           o_ref):
    inv = 1.0 / jnp.maximum(d0_ref[...] + d1_ref[...], 1.0)
    hn = (s0_ref[...] + s1_ref[...]) * inv
    h = h_ref[...]
    o = (jnp.dot(h, ws_ref[...], preferred_element_type=jnp.float32)
         + jnp.dot(hn, wn_ref[...], preferred_element_type=jnp.float32)
         + b_ref[...])
    o_ref[...] = jnp.maximum(o, 0.0) if relu else o
  return body


def _tc_comb(h, s0, s1, d0, d1, ws, wn, b, relu):
  n, d = h.shape
  do = ws.shape[1]
  return pl.pallas_call(
      _make_comb_body(relu),
      grid=(n // BR,),
      in_specs=[
          pl.BlockSpec((BR, d), lambda i: (i, 0)),
          pl.BlockSpec((BR, d), lambda i: (i, 0)),
          pl.BlockSpec((BR, d), lambda i: (i, 0)),
          pl.BlockSpec((BR, 1), lambda i: (i, 0)),
          pl.BlockSpec((BR, 1), lambda i: (i, 0)),
          pl.BlockSpec((d, do), lambda i: (0, 0)),
          pl.BlockSpec((d, do), lambda i: (0, 0)),
          pl.BlockSpec((1, do), lambda i: (0, 0)),
      ],
      out_specs=pl.BlockSpec((BR, do), lambda i: (i, 0)),
      out_shape=jax.ShapeDtypeStruct((n, do), jnp.float32),
  )(h, s0, s1, d0, d1, ws, wn, b)


def kernel(x, edge_index, edge_weight,
           W_self1, W_neigh1, b1,
           W_self2, W_neigh2, b2,
           W_self3, W_neigh3, b3):
  src = edge_index[0].astype(jnp.int32)
  dst = edge_index[1].astype(jnp.int32)

  p1, degp = _sc_agg(True)(x, src, dst)
  degf = degp.reshape(NC, NPAD)
  d0 = degf[0, :N].reshape(N, 1)
  d1 = degf[1, :N].reshape(N, 1)

  h1 = _tc_comb(x, p1[0, :N], p1[1, :N], d0, d1,
                W_self1, W_neigh1, b1.reshape(1, -1), relu=True)
  p2 = _sc_agg(False)(h1, src, dst)
  h2 = _tc_comb(h1, p2[0, :N], p2[1, :N], d0, d1,
                W_self2, W_neigh2, b2.reshape(1, -1), relu=True)
  p3 = _sc_agg(False)(h2, src, dst)
  out = _tc_comb(h2, p3[0, :N], p3[1, :N], d0, d1,
                 W_self3, W_neigh3, b3.reshape(1, -1), relu=False)
  return out
